# baseline (speedup 1.0000x reference)
import functools

import jax
import jax.numpy as jnp
from jax import lax
from jax.experimental import pallas as pl
from jax.experimental.pallas import tpu as pltpu
from jax.experimental.pallas import tpu_sc as plsc

E = 320000
D = 128
S = 10000

NC = 2
NS = 16
NW = NC * NS

BLK = 64
NBUF = 3
NB_LO = 156
NB_HI = 158

STG = 192
FLUSH_AT = STG - BLK
QROWS = 32
NQMAX = STG // QROWS

ROWS_MAIN = 632
ROWS_LAST = S - 15 * ROWS_MAIN
LANES = 16

_mesh = plsc.VectorSubcoreMesh(core_axis_name="c", subcore_axis_name="s")


@functools.partial(
    pl.kernel,
    mesh=_mesh,
    compiler_params=pltpu.CompilerParams(needs_layout_passes=False),
    out_type=jax.ShapeDtypeStruct((NC, S, D), jnp.float32),
    scratch_types=[
        pltpu.VMEM((NBUF, BLK, D), jnp.float32),
        pltpu.VMEM((BLK,), jnp.int32),
        pltpu.VMEM((BLK,), jnp.int32),
        pltpu.VMEM((BLK,), jnp.int32),
        pltpu.VMEM((STG, D), jnp.float32),
        pltpu.VMEM((BLK,), jnp.int32),
        pltpu.VMEM((NQMAX, QROWS), jnp.int32),
        pltpu.SMEM((1,), jnp.int32),
        pltpu.SemaphoreType.DMA,
        pltpu.SemaphoreType.DMA,
        pltpu.SemaphoreType.DMA,
        pltpu.VMEM_SHARED((S, D), jnp.float32),
    ],
)
def _sc_segment_sum(x_hbm, idx_hbm, out_hbm, x_v, i0, i1, i2,
                    staging, rankbuf, useg, count_ref,
                    sem0, sem1, sem2, accum):
    c = lax.axis_index("c")
    s = lax.axis_index("s")
    wid = c * NS + s

    ib = [i0, i1, i2]
    sems = [sem0, sem1, sem2]

    iota16 = jax.lax.iota(jnp.int32, LANES)
    cols = [iota16 + 16 * k for k in range(D // LANES)]

    base = jnp.where(wid < 4, wid * (NB_HI * BLK),
                     4 * (NB_HI * BLK) + (wid - 4) * (NB_LO * BLK))
    nb = jnp.where(wid < 4, NB_HI, NB_LO)

    zero16 = jnp.zeros((LANES,), jnp.float32)

    def zstg(r, carry):
        for k in range(D // LANES):
            staging[r, pl.ds(k * LANES, LANES)] = zero16
        return carry

    lax.fori_loop(0, STG, zstg, 0)
    zero16i = jnp.zeros((LANES,), jnp.int32)
    for q in range(NQMAX):
        for h in range(QROWS // LANES):
            useg[q, pl.ds(h * LANES, LANES)] = zero16i
    count_ref[0] = 0

    row0 = pl.multiple_of(s * ROWS_MAIN, 8)

    @pl.when(s < 15)
    def _():
        for i in range(3):
            pltpu.sync_copy(
                staging.at[pl.ds(0, STG)],
                accum.at[pl.ds(pl.multiple_of(row0 + i * STG, 8), STG)],
            )
        pltpu.sync_copy(
            staging.at[pl.ds(0, ROWS_MAIN - 3 * STG)],
            accum.at[pl.ds(pl.multiple_of(row0 + 3 * STG, 8),
                           ROWS_MAIN - 3 * STG)],
        )

    @pl.when(s == 15)
    def _():
        for i in range(2):
            pltpu.sync_copy(
                staging.at[pl.ds(0, STG)],
                accum.at[pl.ds(pl.multiple_of(row0 + i * STG, 8), STG)],
            )
        pltpu.sync_copy(
            staging.at[pl.ds(0, ROWS_LAST - 2 * STG)],
            accum.at[pl.ds(pl.multiple_of(row0 + 2 * STG, 8),
                           ROWS_LAST - 2 * STG)],
        )

    def load_descs(g, b):
        off = pl.multiple_of(base + g * BLK, 8)
        return [
            pltpu.make_async_copy(idx_hbm.at[pl.ds(off, BLK)], ib[b], sems[b]),
            pltpu.make_async_copy(x_hbm.at[pl.ds(off, BLK)], x_v.at[b], sems[b]),
        ]

    def start_load(g, b):
        for d in load_descs(g, b):
            d.start()

    def wait_load(g, b):
        for d in load_descs(g, b):
            d.wait()

    def flush():
        cnt = count_ref[0]
        nq = (cnt + (QROWS - 1)) >> 5

        def fq(q, carry):
            pltpu.sync_copy(
                staging.at[pl.ds(pl.multiple_of(q * QROWS, 8), QROWS)],
                accum.at[useg.at[q]],
                add=True,
            )
            return carry

        lax.fori_loop(0, nq, fq, 0)

        def zq(q, carry):
            for r in range(QROWS):
                for k in range(D // LANES):
                    staging[q * QROWS + r, pl.ds(k * LANES, LANES)] = zero16
            return carry

        lax.fori_loop(0, nq, zq, 0)
        count_ref[0] = 0

    def process(b):
        @pl.when(count_ref[0] > FLUSH_AT)
        def _():
            flush()

        cnt = count_ref[0]
        total = jnp.int32(0)
        for m in range(BLK // LANES):
            iv = ib[b][pl.ds(m * LANES, LANES)]
            if m == 0:
                pidx = jnp.maximum(iota16 - 1, 0)
            else:
                pidx = iota16 + (m * LANES - 1)
            pv = plsc.load_gather(ib[b], [pidx])
            bnd = jnp.where(iv != pv, jnp.int32(1), jnp.int32(0))
            if m == 0:
                bnd = jnp.where(iota16 == 0, jnp.int32(1), bnd)
            lr = plsc.cumsum(bnd)
            slot = lr + (cnt + total - 1)
            rankbuf[pl.ds(m * LANES, LANES)] = slot
            plsc.store_scatter(useg, [slot >> 5, slot & (QROWS - 1)], iv)
            total = total + jnp.sum(bnd)
        count_ref[0] = cnt + total

        def frow(r, carry):
            rowslot = plsc.load_gather(rankbuf, [jnp.full((LANES,), r, jnp.int32)])
            for k in range(D // LANES):
                xk = x_v[b, r, pl.ds(k * LANES, LANES)]
                plsc.addupdate_scatter(staging, [rowslot, cols[k]], xk)
            return carry

        lax.fori_loop(0, BLK, frow, 0)

    start_load(0, 0)
    start_load(1, 1)
    plsc.subcore_barrier()

    def body(jo, carry):
        for b in range(NBUF):
            g = jo * NBUF + b

            @pl.when(g + 2 < nb)
            def _():
                start_load(g + 2, (b + 2) % NBUF)

            wait_load(g, b)
            process(b)
        return carry

    lax.fori_loop(0, NB_LO // NBUF, body, 0)

    @pl.when(nb == NB_HI)
    def _():
        for g in (NB_LO, NB_LO + 1):
            wait_load(g, g % NBUF)
            process(g % NBUF)

    @pl.when(count_ref[0] > 0)
    def _():
        flush()

    plsc.subcore_barrier()

    @pl.when(s < 15)
    def _():
        pltpu.sync_copy(
            accum.at[pl.ds(row0, ROWS_MAIN)],
            out_hbm.at[c].at[pl.ds(row0, ROWS_MAIN)],
        )

    @pl.when(s == 15)
    def _():
        pltpu.sync_copy(
            accum.at[pl.ds(row0, ROWS_LAST)],
            out_hbm.at[c].at[pl.ds(row0, ROWS_LAST)],
        )


_RB = 1000


def _combine_body(p_ref, o_ref):
    o_ref[...] = p_ref[0] + p_ref[1]


def _combine(partials):
    return pl.pallas_call(
        _combine_body,
        grid=(S // _RB,),
        in_specs=[pl.BlockSpec((NC, _RB, D), lambda i: (0, i, 0))],
        out_specs=pl.BlockSpec((_RB, D), lambda i: (i, 0)),
        out_shape=jax.ShapeDtypeStruct((S, D), jnp.float32),
    )(partials)


def kernel(x, index):
    idx32 = index.astype(jnp.int32)
    partials = _sc_segment_sum(x, idx32)
    return _combine(partials)


# speedup vs baseline: 1.6044x; 1.6044x over previous
import functools

import jax
import jax.numpy as jnp
from jax import lax
from jax.experimental import pallas as pl
from jax.experimental.pallas import tpu as pltpu
from jax.experimental.pallas import tpu_sc as plsc

E = 320000
D = 128
S = 10000

NC = 2
NS = 16
NW = NC * NS

BLK = 64
NBUF = 3
NB_LO = 156
NB_HI = 158

STG = 192
FLUSH_AT = STG - BLK
QROWS = 32
NQMAX = STG // QROWS

ROWS_MAIN = 632
ROWS_LAST = S - 15 * ROWS_MAIN
LANES = 16

_mesh = plsc.VectorSubcoreMesh(core_axis_name="c", subcore_axis_name="s")


@functools.partial(
    pl.kernel,
    mesh=_mesh,
    compiler_params=pltpu.CompilerParams(needs_layout_passes=False),
    out_type=jax.ShapeDtypeStruct((NC, S, D), jnp.float32),
    scratch_types=[
        pltpu.VMEM((NBUF, BLK, D), jnp.float32),
        pltpu.VMEM((BLK,), jnp.int32),
        pltpu.VMEM((BLK,), jnp.int32),
        pltpu.VMEM((BLK,), jnp.int32),
        pltpu.VMEM((STG, D), jnp.float32),
        pltpu.VMEM((BLK,), jnp.int32),
        pltpu.VMEM((NQMAX, QROWS), jnp.int32),
        pltpu.SMEM((1,), jnp.int32),
        pltpu.SemaphoreType.DMA,
        pltpu.SemaphoreType.DMA,
        pltpu.SemaphoreType.DMA,
        pltpu.VMEM_SHARED((S, D), jnp.float32),
    ],
)
def _sc_segment_sum(x_hbm, idx_hbm, out_hbm, x_v, i0, i1, i2,
                    staging, rankbuf, useg, count_ref,
                    sem0, sem1, sem2, accum):
    c = lax.axis_index("c")
    s = lax.axis_index("s")
    wid = c * NS + s

    ib = [i0, i1, i2]
    sems = [sem0, sem1, sem2]

    iota16 = jax.lax.iota(jnp.int32, LANES)
    cols = [iota16 + 16 * k for k in range(D // LANES)]

    base = jnp.where(wid < 4, wid * (NB_HI * BLK),
                     4 * (NB_HI * BLK) + (wid - 4) * (NB_LO * BLK))
    nb = jnp.where(wid < 4, NB_HI, NB_LO)

    zero16 = jnp.zeros((LANES,), jnp.float32)

    @plsc.parallel_loop(0, STG, unroll=8)
    def _(r):
        for k in range(D // LANES):
            staging[r, pl.ds(k * LANES, LANES)] = zero16
    zero16i = jnp.zeros((LANES,), jnp.int32)
    for q in range(NQMAX):
        for h in range(QROWS // LANES):
            useg[q, pl.ds(h * LANES, LANES)] = zero16i
    count_ref[0] = 0

    row0 = pl.multiple_of(s * ROWS_MAIN, 8)

    @pl.when(s < 15)
    def _():
        for i in range(3):
            pltpu.sync_copy(
                staging.at[pl.ds(0, STG)],
                accum.at[pl.ds(pl.multiple_of(row0 + i * STG, 8), STG)],
            )
        pltpu.sync_copy(
            staging.at[pl.ds(0, ROWS_MAIN - 3 * STG)],
            accum.at[pl.ds(pl.multiple_of(row0 + 3 * STG, 8),
                           ROWS_MAIN - 3 * STG)],
        )

    @pl.when(s == 15)
    def _():
        for i in range(2):
            pltpu.sync_copy(
                staging.at[pl.ds(0, STG)],
                accum.at[pl.ds(pl.multiple_of(row0 + i * STG, 8), STG)],
            )
        pltpu.sync_copy(
            staging.at[pl.ds(0, ROWS_LAST - 2 * STG)],
            accum.at[pl.ds(pl.multiple_of(row0 + 2 * STG, 8),
                           ROWS_LAST - 2 * STG)],
        )

    def load_descs(g, b):
        off = pl.multiple_of(base + g * BLK, 8)
        return [
            pltpu.make_async_copy(idx_hbm.at[pl.ds(off, BLK)], ib[b], sems[b]),
            pltpu.make_async_copy(x_hbm.at[pl.ds(off, BLK)], x_v.at[b], sems[b]),
        ]

    def start_load(g, b):
        for d in load_descs(g, b):
            d.start()

    def wait_load(g, b):
        for d in load_descs(g, b):
            d.wait()

    def flush():
        cnt = count_ref[0]
        nq = (cnt + (QROWS - 1)) >> 5

        def fq(q, carry):
            pltpu.sync_copy(
                staging.at[pl.ds(pl.multiple_of(q * QROWS, 8), QROWS)],
                accum.at[useg.at[q]],
                add=True,
            )
            return carry

        lax.fori_loop(0, nq, fq, 0)

        @plsc.parallel_loop(0, nq * QROWS, unroll=8)
        def _(r):
            for k in range(D // LANES):
                staging[r, pl.ds(k * LANES, LANES)] = zero16
        count_ref[0] = 0

    def process(b):
        @pl.when(count_ref[0] > FLUSH_AT)
        def _():
            flush()

        cnt = count_ref[0]
        total = jnp.int32(0)
        for m in range(BLK // LANES):
            iv = ib[b][pl.ds(m * LANES, LANES)]
            if m == 0:
                pidx = jnp.maximum(iota16 - 1, 0)
            else:
                pidx = iota16 + (m * LANES - 1)
            pv = plsc.load_gather(ib[b], [pidx])
            bnd = jnp.where(iv != pv, jnp.int32(1), jnp.int32(0))
            if m == 0:
                bnd = jnp.where(iota16 == 0, jnp.int32(1), bnd)
            lr = plsc.cumsum(bnd)
            slot = lr + (cnt + total - 1)
            rankbuf[pl.ds(m * LANES, LANES)] = slot
            plsc.store_scatter(useg, [slot >> 5, slot & (QROWS - 1)], iv)
            total = total + jnp.sum(bnd)
        count_ref[0] = cnt + total

        @plsc.parallel_loop(0, BLK, unroll=8)
        def _(r):
            rowslot = plsc.load_gather(rankbuf, [jnp.full((LANES,), r, jnp.int32)])
            for k in range(D // LANES):
                xk = x_v[b, r, pl.ds(k * LANES, LANES)]
                plsc.addupdate_scatter(staging, [rowslot, cols[k]], xk)

    start_load(0, 0)
    start_load(1, 1)
    plsc.subcore_barrier()

    def body(jo, carry):
        for b in range(NBUF):
            g = jo * NBUF + b

            @pl.when(g + 2 < nb)
            def _():
                start_load(g + 2, (b + 2) % NBUF)

            wait_load(g, b)
            process(b)
        return carry

    lax.fori_loop(0, NB_LO // NBUF, body, 0)

    @pl.when(nb == NB_HI)
    def _():
        for g in (NB_LO, NB_LO + 1):
            wait_load(g, g % NBUF)
            process(g % NBUF)

    @pl.when(count_ref[0] > 0)
    def _():
        flush()

    plsc.subcore_barrier()

    @pl.when(s < 15)
    def _():
        pltpu.sync_copy(
            accum.at[pl.ds(row0, ROWS_MAIN)],
            out_hbm.at[c].at[pl.ds(row0, ROWS_MAIN)],
        )

    @pl.when(s == 15)
    def _():
        pltpu.sync_copy(
            accum.at[pl.ds(row0, ROWS_LAST)],
            out_hbm.at[c].at[pl.ds(row0, ROWS_LAST)],
        )


_RB = 1000


def _combine_body(p_ref, o_ref):
    o_ref[...] = p_ref[0] + p_ref[1]


def _combine(partials):
    return pl.pallas_call(
        _combine_body,
        grid=(S // _RB,),
        in_specs=[pl.BlockSpec((NC, _RB, D), lambda i: (0, i, 0))],
        out_specs=pl.BlockSpec((_RB, D), lambda i: (i, 0)),
        out_shape=jax.ShapeDtypeStruct((S, D), jnp.float32),
    )(partials)


def kernel(x, index):
    idx32 = index.astype(jnp.int32)
    partials = _sc_segment_sum(x, idx32)
    return _combine(partials)


# speedup vs baseline: 3.0329x; 1.8904x over previous
import functools

import jax
import jax.numpy as jnp
from jax import lax
from jax.experimental import pallas as pl
from jax.experimental.pallas import tpu as pltpu
from jax.experimental.pallas import tpu_sc as plsc

E = 320000
D = 128
S = 10000
SPAD = 10112

NC = 2
NS = 16
NW = NC * NS

BLK = 128
NBUF = 3
NB_LO = 78
NB_HI = 79

ROWS_PER_TILE = SPAD // NS
LANES = 16

_mesh = plsc.VectorSubcoreMesh(core_axis_name="c", subcore_axis_name="s")


@functools.partial(
    pl.kernel,
    mesh=_mesh,
    out_type=jax.ShapeDtypeStruct((NC, SPAD, D), jnp.float32),
    scratch_types=[
        pltpu.VMEM((NBUF, BLK, D), jnp.float32),
        pltpu.VMEM((BLK,), jnp.int32),
        pltpu.VMEM((BLK,), jnp.int32),
        pltpu.VMEM((BLK,), jnp.int32),
        pltpu.SemaphoreType.DMA,
        pltpu.SemaphoreType.DMA,
        pltpu.SemaphoreType.DMA,
        pltpu.VMEM_SHARED((SPAD, D), jnp.float32),
    ],
)
def _sc_segment_sum(x_hbm, idx_hbm, out_hbm, x_v,
                    i0, i1, i2, sem0, sem1, sem2, accum):
    c = lax.axis_index("c")
    s = lax.axis_index("s")
    wid = c * NS + s

    ib = [i0, i1, i2]
    sems = [sem0, sem1, sem2]

    base = jnp.where(wid < 4, wid * (NB_HI * BLK),
                     4 * (NB_HI * BLK) + (wid - 4) * (NB_LO * BLK))
    nb = jnp.where(wid < 4, NB_HI, NB_LO)

    zero16 = jnp.zeros((LANES,), jnp.float32)

    def zrow(r, carry):
        for k in range(D // LANES):
            x_v[0, r, pl.ds(k * LANES, LANES)] = zero16
        return carry

    lax.fori_loop(0, BLK, zrow, 0)

    row0 = pl.multiple_of(s * ROWS_PER_TILE, 8)
    for i in range(4):
        pltpu.sync_copy(
            x_v.at[0],
            accum.at[pl.ds(pl.multiple_of(row0 + i * BLK, 8), BLK)],
        )
    pltpu.sync_copy(
        x_v.at[0, pl.ds(0, ROWS_PER_TILE - 4 * BLK)],
        accum.at[pl.ds(pl.multiple_of(row0 + 4 * BLK, 8), ROWS_PER_TILE - 4 * BLK)],
    )

    def load_descs(g, b):
        off = pl.multiple_of(base + g * BLK, 8)
        return [
            pltpu.make_async_copy(idx_hbm.at[pl.ds(off, BLK)], ib[b], sems[b]),
            pltpu.make_async_copy(x_hbm.at[pl.ds(off, BLK)], x_v.at[b], sems[b]),
        ]

    def start_load(g, b):
        for d in load_descs(g, b):
            d.start()

    def wait_load(g, b):
        for d in load_descs(g, b):
            d.wait()

    def scatter(b):
        pltpu.sync_copy(x_v.at[b], accum.at[ib[b]], add=True)

    start_load(0, 0)
    start_load(1, 1)
    plsc.subcore_barrier()

    def body(jo, carry):
        for b in range(NBUF):
            g = jo * NBUF + b

            @pl.when(g + 2 < nb)
            def _():
                start_load(g + 2, (b + 2) % NBUF)

            wait_load(g, b)
            scatter(b)
        return carry

    lax.fori_loop(0, NB_LO // NBUF, body, 0)

    @pl.when(nb == NB_HI)
    def _():
        wait_load(NB_LO, NB_LO % NBUF)
        scatter(NB_LO % NBUF)

    plsc.subcore_barrier()

    pltpu.sync_copy(
        accum.at[pl.ds(row0, ROWS_PER_TILE)],
        out_hbm.at[c].at[pl.ds(row0, ROWS_PER_TILE)],
    )


_RB = 1000


def _combine_body(p_ref, o_ref):
    o_ref[...] = p_ref[0] + p_ref[1]


def _combine(partials):
    return pl.pallas_call(
        _combine_body,
        grid=(S // _RB,),
        in_specs=[pl.BlockSpec((NC, _RB, D), lambda i: (0, i, 0))],
        out_specs=pl.BlockSpec((_RB, D), lambda i: (i, 0)),
        out_shape=jax.ShapeDtypeStruct((S, D), jnp.float32),
    )(partials)


def kernel(x, index):
    idx32 = index.astype(jnp.int32)
    partials = _sc_segment_sum(x, idx32)
    return _combine(partials)
